# baseline (speedup 1.0000x reference)
import jax
import jax.numpy as jnp
from jax import lax
from jax.experimental import pallas as pl
from jax.experimental.pallas import tpu as pltpu
from jax.experimental.pallas import tpu_sc as plsc

_VOCAB = 100000
_D = 64
_B, _S, _W = 1024, 20, 50
_N = _B * _S * _W
_SW = _S * _W

_NC = 2
_NS = 16
_NW = _NC * _NS

_PER_W = _N // _NW
_CH = 128
_N_CH = _PER_W // _CH
_K = 5
_GROUP = _K * _CH
_NG = _N_CH // _K


def _body(idx_hbm, table_hbm, out_hbm, idx_v, rows0, rows1,
          g0s, g1s, w0s, w1s):
    wid = lax.axis_index("s") * _NC + lax.axis_index("c")
    base = wid * _PER_W
    pltpu.sync_copy(idx_hbm.at[wid], idx_v)

    def fire_g(g, rows, sem):
        for b in range(_K):
            pltpu.async_copy(table_hbm.at[idx_v.at[g * _K + b]],
                             rows.at[pl.ds(b * _CH, _CH)], sem)

    def drain_g(g, rows, sem):
        for b in range(_K):
            pltpu.make_async_copy(table_hbm.at[idx_v.at[g * _K + b]],
                                  rows.at[pl.ds(b * _CH, _CH)], sem).wait()

    def fire_w(g, rows, sem):
        pltpu.async_copy(rows, out_hbm.at[pl.ds(base + g * _GROUP, _GROUP)],
                         sem)

    def drain_w(g, rows, sem):
        pltpu.make_async_copy(rows,
                              out_hbm.at[pl.ds(base + g * _GROUP, _GROUP)],
                              sem).wait()

    fire_g(0, rows0, g0s)
    drain_g(0, rows0, g0s)
    fire_g(1, rows1, g1s)
    fire_w(0, rows0, w0s)

    def pair(t, carry):
        ga = 2 * t + 1
        gb = 2 * t + 2
        drain_g(ga, rows1, g1s)
        drain_w(ga - 1, rows0, w0s)
        fire_g(gb, rows0, g0s)
        fire_w(ga, rows1, w1s)
        drain_g(gb, rows0, g0s)
        drain_w(gb - 1, rows1, w1s)
        fire_g(gb + 1, rows1, g1s)
        fire_w(gb, rows0, w0s)
        return carry

    lax.fori_loop(0, _NG // 2 - 1, pair, 0)

    drain_g(_NG - 1, rows1, g1s)
    drain_w(_NG - 2, rows0, w0s)
    fire_w(_NG - 1, rows1, w1s)
    drain_w(_NG - 1, rows1, w1s)


@jax.jit
def _gather(idx, table):
    mesh = plsc.VectorSubcoreMesh(
        core_axis_name="c", subcore_axis_name="s",
        num_cores=_NC, num_subcores=_NS)
    f = pl.kernel(
        _body,
        out_type=jax.ShapeDtypeStruct((_N, _D), jnp.float32),
        mesh=mesh,
        scratch_types=[
            pltpu.VMEM((_N_CH, _CH), jnp.int32),
            pltpu.VMEM((_GROUP, _D), jnp.float32),
            pltpu.VMEM((_GROUP, _D), jnp.float32),
            pltpu.SemaphoreType.DMA,
            pltpu.SemaphoreType.DMA,
            pltpu.SemaphoreType.DMA,
            pltpu.SemaphoreType.DMA,
        ],
        compiler_params=pltpu.CompilerParams(use_tc_tiling_on_sc=False),
    )
    return f(idx, table)


def _tbody(z_ref, y_ref):
    z = z_ref[0]
    y_ref[0, 0, :, : _B // 2] = z[:, :_D].T
    y_ref[0, 0, :, _B // 2:] = z[:, _D:].T


@jax.jit
def _transpose(z):
    return pl.pallas_call(
        _tbody,
        grid=(_SW,),
        in_specs=[pl.BlockSpec((1, _B // 2, 2 * _D), lambda sw: (sw, 0, 0))],
        out_specs=pl.BlockSpec((1, 1, _D, _B),
                               lambda sw: (sw // _W, sw % _W, 0, 0)),
        out_shape=jax.ShapeDtypeStruct((_S, _W, _D, _B), jnp.float32),
    )(z)


def kernel(X, pad_vector, embedding_table):
    idx = (X.transpose(1, 2, 0).reshape(_S, _W, 2, _B // 2)
           .transpose(0, 1, 3, 2).reshape(_NW, _N_CH, _CH).astype(jnp.int32))
    rows = _gather(idx, embedding_table)
    y = _transpose(rows.reshape(_SW, _B // 2, 2 * _D))
    return jnp.transpose(y, (3, 0, 1, 2))


# speedup vs baseline: 1.0806x; 1.0806x over previous
import jax
import jax.numpy as jnp
from jax import lax
from jax.experimental import pallas as pl
from jax.experimental.pallas import tpu as pltpu
from jax.experimental.pallas import tpu_sc as plsc

_VOCAB = 100000
_D = 64
_B, _S, _W = 1024, 20, 50
_N = _B * _S * _W
_SW = _S * _W
_H = _B // 2
_NL = _N // 2

_NC = 2
_NS = 16
_NW = _NC * _NS
_LPW = _NL // _NW
_CH = 128
_K = 5
_GR = _K * _CH
_GL = _GR // 2
_NG = _LPW // _GL


def _body(idxa_hbm, idxb_hbm, table_hbm, out_hbm, idxa_v, idxb_v,
          idxg0, idxg1, rows0, rows1, g0s, g1s, w0s, w1s):
    wid = lax.axis_index("s") * _NC + lax.axis_index("c")
    lbase = wid * _LPW
    pltpu.sync_copy(idxa_hbm.at[pl.ds(lbase, _LPW)], idxa_v)
    pltpu.sync_copy(idxb_hbm.at[pl.ds(lbase, _LPW)], idxb_v)

    two_iota = 2 * lax.iota(jnp.int32, 16)

    def build_idx(g, idxg):
        for b in range(_K):
            for i in range(4):
                off = g * _GL + b * 64 + i * 16
                pos = two_iota + (b * _CH + 32 * i)
                plsc.store_scatter(idxg, [pos], idxa_v[pl.ds(off, 16)])
                plsc.store_scatter(idxg, [pos + 1], idxb_v[pl.ds(off, 16)])

    def fire_g(idxg, rows, sem):
        for b in range(_K):
            pltpu.async_copy(table_hbm.at[idxg.at[pl.ds(b * _CH, _CH)]],
                             rows.at[pl.ds(b * _CH, _CH)], sem)

    def drain_g(idxg, rows, sem):
        for b in range(_K):
            pltpu.make_async_copy(table_hbm.at[idxg.at[pl.ds(b * _CH, _CH)]],
                                  rows.at[pl.ds(b * _CH, _CH)], sem).wait()

    def fire_w(g, rows, sem):
        pltpu.async_copy(rows, out_hbm.at[pl.ds(2 * (lbase + g * _GL), _GR)],
                         sem)

    def drain_w(g, rows, sem):
        pltpu.make_async_copy(rows,
                              out_hbm.at[pl.ds(2 * (lbase + g * _GL), _GR)],
                              sem).wait()

    build_idx(0, idxg0)
    fire_g(idxg0, rows0, g0s)
    build_idx(1, idxg1)
    drain_g(idxg0, rows0, g0s)
    fire_g(idxg1, rows1, g1s)
    fire_w(0, rows0, w0s)

    def pair(t, carry):
        ga = 2 * t + 1
        gb = 2 * t + 2
        drain_g(idxg1, rows1, g1s)
        drain_w(ga - 1, rows0, w0s)
        build_idx(gb, idxg0)
        fire_g(idxg0, rows0, g0s)
        fire_w(ga, rows1, w1s)
        drain_g(idxg0, rows0, g0s)
        drain_w(gb - 1, rows1, w1s)
        build_idx(gb + 1, idxg1)
        fire_g(idxg1, rows1, g1s)
        fire_w(gb, rows0, w0s)
        return carry

    lax.fori_loop(0, _NG // 2 - 1, pair, 0)

    drain_g(idxg1, rows1, g1s)
    drain_w(_NG - 2, rows0, w0s)
    fire_w(_NG - 1, rows1, w1s)
    drain_w(_NG - 1, rows1, w1s)


@jax.jit
def _gather(idxa, idxb, table):
    mesh = plsc.VectorSubcoreMesh(
        core_axis_name="c", subcore_axis_name="s",
        num_cores=_NC, num_subcores=_NS)
    f = pl.kernel(
        _body,
        out_type=jax.ShapeDtypeStruct((_N, _D), jnp.float32),
        mesh=mesh,
        scratch_types=[
            pltpu.VMEM((_LPW,), jnp.int32),
            pltpu.VMEM((_LPW,), jnp.int32),
            pltpu.VMEM((_GR,), jnp.int32),
            pltpu.VMEM((_GR,), jnp.int32),
            pltpu.VMEM((_GR, _D), jnp.float32),
            pltpu.VMEM((_GR, _D), jnp.float32),
            pltpu.SemaphoreType.DMA,
            pltpu.SemaphoreType.DMA,
            pltpu.SemaphoreType.DMA,
            pltpu.SemaphoreType.DMA,
        ],
        compiler_params=pltpu.CompilerParams(use_tc_tiling_on_sc=False,
                                             needs_layout_passes=False),
    )
    return f(idxa, idxb, table)


def _tbody(z_ref, y_ref):
    z = z_ref[0]
    eye = jnp.eye(_D, dtype=jnp.float32)
    nt = (((1,), (1,)), ((), ()))
    y_ref[0, 0, :, :_H] = lax.dot_general(
        eye, z[:, :_D], nt, precision=lax.Precision.HIGHEST,
        preferred_element_type=jnp.float32)
    y_ref[0, 0, :, _H:] = lax.dot_general(
        eye, z[:, _D:], nt, precision=lax.Precision.HIGHEST,
        preferred_element_type=jnp.float32)


@jax.jit
def _transpose(z):
    return pl.pallas_call(
        _tbody,
        grid=(_SW,),
        in_specs=[pl.BlockSpec((1, _H, 2 * _D), lambda sw: (sw, 0, 0))],
        out_specs=pl.BlockSpec((1, 1, _D, _B),
                               lambda sw: (sw // _W, sw % _W, 0, 0)),
        out_shape=jax.ShapeDtypeStruct((_S, _W, _D, _B), jnp.float32),
    )(z)


def kernel(X, pad_vector, embedding_table):
    xt = X.transpose(1, 2, 0).astype(jnp.int32)
    idxa = xt[:, :, :_H].reshape(_NL)
    idxb = xt[:, :, _H:].reshape(_NL)
    rows = _gather(idxa, idxb, embedding_table)
    y = _transpose(rows.reshape(_SW, _H, 2 * _D))
    return jnp.transpose(y, (3, 0, 1, 2))


# speedup vs baseline: 1.3703x; 1.2681x over previous
import jax
import jax.numpy as jnp
from jax import lax
from jax.experimental import pallas as pl
from jax.experimental.pallas import tpu as pltpu
from jax.experimental.pallas import tpu_sc as plsc

_VOCAB = 100000
_D = 64
_B = 1024 * 20 * 50

_NC = 2
_NS = 16
_NW = _NC * _NS

_PER_W = _B // _NW
_CH = 128
_N_CH = _PER_W // _CH
_K = 5
_GROUP = _K * _CH
_NG = _N_CH // _K


def _body(idx_hbm, table_hbm, out_hbm, idx_v, rows0, rows1,
          g0s, g1s, w0s, w1s):
    wid = lax.axis_index("s") * _NC + lax.axis_index("c")
    base = wid * _PER_W
    pltpu.sync_copy(idx_hbm.at[wid], idx_v)

    def fire_g(g, rows, sem):
        for b in range(_K):
            pltpu.async_copy(table_hbm.at[idx_v.at[g * _K + b]],
                             rows.at[pl.ds(b * _CH, _CH)], sem)

    def drain_g(g, rows, sem):
        for b in range(_K):
            pltpu.make_async_copy(table_hbm.at[idx_v.at[g * _K + b]],
                                  rows.at[pl.ds(b * _CH, _CH)], sem).wait()

    def fire_w(g, rows, sem):
        pltpu.async_copy(rows, out_hbm.at[pl.ds(base + g * _GROUP, _GROUP)],
                         sem)

    def drain_w(g, rows, sem):
        pltpu.make_async_copy(rows,
                              out_hbm.at[pl.ds(base + g * _GROUP, _GROUP)],
                              sem).wait()

    fire_g(0, rows0, g0s)
    drain_g(0, rows0, g0s)
    fire_g(1, rows1, g1s)
    fire_w(0, rows0, w0s)

    def pair(t, carry):
        ga = 2 * t + 1
        gb = 2 * t + 2
        drain_g(ga, rows1, g1s)
        drain_w(ga - 1, rows0, w0s)
        fire_g(gb, rows0, g0s)
        fire_w(ga, rows1, w1s)
        drain_g(gb, rows0, g0s)
        drain_w(gb - 1, rows1, w1s)
        fire_g(gb + 1, rows1, g1s)
        fire_w(gb, rows0, w0s)
        return carry

    lax.fori_loop(0, _NG // 2 - 1, pair, 0)

    drain_g(_NG - 1, rows1, g1s)
    drain_w(_NG - 2, rows0, w0s)
    fire_w(_NG - 1, rows1, w1s)
    drain_w(_NG - 1, rows1, w1s)


@jax.jit
def _gather(idx, table):
    mesh = plsc.VectorSubcoreMesh(
        core_axis_name="c", subcore_axis_name="s",
        num_cores=_NC, num_subcores=_NS)
    f = pl.kernel(
        _body,
        out_type=jax.ShapeDtypeStruct((_B, _D), jnp.float32),
        mesh=mesh,
        scratch_types=[
            pltpu.VMEM((_N_CH, _CH), jnp.int32),
            pltpu.VMEM((_GROUP, _D), jnp.float32),
            pltpu.VMEM((_GROUP, _D), jnp.float32),
            pltpu.SemaphoreType.DMA,
            pltpu.SemaphoreType.DMA,
            pltpu.SemaphoreType.DMA,
            pltpu.SemaphoreType.DMA,
        ],
        compiler_params=pltpu.CompilerParams(use_tc_tiling_on_sc=False),
    )
    return f(idx, table)


def kernel(X, pad_vector, embedding_table):
    idx = X.reshape(_NW, _N_CH, _CH).astype(jnp.int32)
    out = _gather(idx, embedding_table)
    return out.reshape(X.shape + (_D,))
